# baseline (speedup 1.0000x reference)
import functools

import jax
import jax.numpy as jnp
from jax import lax
from jax.experimental import pallas as pl
from jax.experimental.pallas import tpu as pltpu
from jax.experimental.pallas import tpu_sc as plsc

_MAXD = 128
_H = 16
_T = 2048
_WLEN = 4112
_NW = 32
_ROWS_PER_W = (_H * _T) // _NW
_GROUP = 8
_FIRE = 4


def _sc_bias(wbuf):
    mesh = plsc.VectorSubcoreMesh(core_axis_name="c", subcore_axis_name="s")

    @functools.partial(
        pl.kernel,
        out_type=jax.ShapeDtypeStruct((_H * _T, _T), jnp.float32),
        mesh=mesh,
        scratch_types=[
            pltpu.VMEM((_GROUP, _WLEN), jnp.float32),
            pltpu.SemaphoreType.DMA,
        ],
        compiler_params=pltpu.CompilerParams(use_tc_tiling_on_sc=False),
    )
    def k(wbuf_hbm, out_hbm, w_v, sem):
        cid = lax.axis_index("c")
        sid = lax.axis_index("s")
        wid = sid * 2 + cid
        head = wid // 2
        half = wid % 2

        pltpu.sync_copy(wbuf_hbm.at[head], w_v)

        row0 = head * _T + half * _ROWS_PER_W

        def emit(it, carry):
            copies = []
            for f in range(_FIRE):
                g = it * _FIRE + f
                q = _T - half * _ROWS_PER_W - _GROUP
                src = w_v.at[:, pl.ds(pl.multiple_of(q - g * _GROUP, 8), _T)]
                d_row = pl.multiple_of(row0 + g * _GROUP, 8)
                dst = out_hbm.at[pl.ds(d_row, _GROUP), :]
                copies.append(pltpu.async_copy(src, dst, sem))
            for c in copies:
                c.wait()
            return carry

        lax.fori_loop(0, _ROWS_PER_W // (_GROUP * _FIRE), emit, 0)

    return k(wbuf)


def kernel(T, table):
    m = jnp.arange(_WLEN)
    s = jnp.arange(_GROUP)
    p = m[None, :] + (_GROUP - s)[:, None]
    rows = jnp.clip(p - _T, -_MAXD, _MAXD) + _MAXD
    wbuf = jnp.transpose(table[rows], (2, 0, 1))
    out2d = _sc_bias(wbuf)
    return out2d.reshape(1, _H, _T, _T)


# speedup vs baseline: 1.1594x; 1.1594x over previous
import functools

import jax
import jax.numpy as jnp
from jax import lax
from jax.experimental import pallas as pl
from jax.experimental.pallas import tpu as pltpu
from jax.experimental.pallas import tpu_sc as plsc

_MAXD = 128
_H = 16
_T = 2048
_WLEN = 4112
_NW = 32
_ROWS_PER_W = (_H * _T) // _NW
_FIRE = 32
_TTL = 304
_BAND0 = 1896
_FILL_LO = _BAND0 + 8
_FILL_HI = _BAND0 + 296


def _sc_bias(tt_all):
    mesh = plsc.VectorSubcoreMesh(core_axis_name="c", subcore_axis_name="s")

    @functools.partial(
        pl.kernel,
        out_type=jax.ShapeDtypeStruct((_H * _T * _T,), jnp.float32),
        mesh=mesh,
        scratch_types=[
            pltpu.VMEM((8 * _WLEN,), jnp.float32),
            pltpu.SemaphoreType.DMA,
        ],
    )
    def k(tt_hbm, out_hbm, w_v, sem):
        cid = lax.axis_index("c")
        sid = lax.axis_index("s")
        wid = sid * 2 + cid
        head = wid // 2
        half = wid % 2

        for r in range(8):
            base = r * _WLEN
            src_off = pl.multiple_of((r * _H + head) * _TTL, 8)
            pltpu.sync_copy(
                tt_hbm.at[pl.ds(src_off, _TTL)],
                w_v.at[pl.ds(base + _BAND0, _TTL)],
            )
            c_lo = w_v[pl.ds(base + _BAND0, 16)]
            c_hi = w_v[pl.ds(base + _BAND0 + 288, 16)]

            def fill(g, carry, base=base, c_lo=c_lo, c_hi=c_hi):
                off = pl.multiple_of(g * 16, 16)
                w_v[pl.ds(base + off, 16)] = c_lo
                w_v[pl.ds(base + _FILL_HI + off, 16)] = c_hi
                return carry

            lax.fori_loop(0, _FILL_LO // 16, fill, 0)

        row0 = head * _T + half * _ROWS_PER_W

        def emit(it, carry):
            base8 = _T - half * _ROWS_PER_W - it * _FIRE
            copies = []
            for b in range(_FIRE):
                u = b % 8
                r = (8 - u) % 8
                q8 = base8 - (b - u) - (8 if u else 0)
                src = w_v.at[pl.ds(pl.multiple_of(r * _WLEN + q8, 8), _T)]
                d_off = pl.multiple_of((row0 + it * _FIRE + b) * _T, _T)
                dst = out_hbm.at[pl.ds(d_off, _T)]
                copies.append(pltpu.async_copy(src, dst, sem))
            for c in copies:
                c.wait()
            return carry

        lax.fori_loop(0, _ROWS_PER_W // _FIRE, emit, 0)

    return k(tt_all)


def kernel(T, table):
    j = jnp.arange(_TTL)
    r = jnp.arange(8)
    rows = jnp.clip(j[None, :] + r[:, None] - 24, 0, 2 * _MAXD)
    tt_all = jnp.transpose(table[rows], (0, 2, 1)).reshape(-1)
    out2d = _sc_bias(tt_all)
    return out2d.reshape(1, _H, _T, _T)


# speedup vs baseline: 1.1637x; 1.0037x over previous
import functools

import jax
import jax.numpy as jnp
from jax import lax
from jax.experimental import pallas as pl
from jax.experimental.pallas import tpu as pltpu
from jax.experimental.pallas import tpu_sc as plsc

_MAXD = 128
_H = 16
_T = 2048
_WLEN = 4112
_NW = 32
_ROWS_PER_W = (_H * _T) // _NW
_FIRE = 32
_TTL = 304
_BAND0 = 1896
_FILL_LO = _BAND0 + 8
_FILL_HI = _BAND0 + 296


def _sc_bias(tt_all):
    mesh = plsc.VectorSubcoreMesh(core_axis_name="c", subcore_axis_name="s")

    @functools.partial(
        pl.kernel,
        out_type=jax.ShapeDtypeStruct((1, _H, _T, _T), jnp.float32),
        mesh=mesh,
        scratch_types=[
            pltpu.VMEM((8 * _WLEN,), jnp.float32),
            pltpu.SemaphoreType.DMA,
        ],
        compiler_params=pltpu.CompilerParams(use_tc_tiling_on_sc=False),
    )
    def k(tt_hbm, out_hbm, w_v, sem):
        cid = lax.axis_index("c")
        sid = lax.axis_index("s")
        wid = sid * 2 + cid
        head = wid // 2
        half = wid % 2

        for r in range(8):
            base = r * _WLEN
            src_off = pl.multiple_of((r * _H + head) * _TTL, 8)
            pltpu.sync_copy(
                tt_hbm.at[pl.ds(src_off, _TTL)],
                w_v.at[pl.ds(base + _BAND0, _TTL)],
            )
            c_lo = w_v[pl.ds(base + _BAND0, 16)]
            c_hi = w_v[pl.ds(base + _BAND0 + 288, 16)]

            def fill(g, carry, base=base, c_lo=c_lo, c_hi=c_hi):
                off = pl.multiple_of(g * 16, 16)
                w_v[pl.ds(base + off, 16)] = c_lo
                w_v[pl.ds(base + _FILL_HI + off, 16)] = c_hi
                return carry

            lax.fori_loop(0, _FILL_LO // 16, fill, 0)

        def emit(it, carry):
            base8 = _T - half * _ROWS_PER_W - it * _FIRE
            copies = []
            for b in range(_FIRE):
                u = b % 8
                r = (8 - u) % 8
                q8 = base8 - (b - u) - (8 if u else 0)
                src = w_v.at[pl.ds(pl.multiple_of(r * _WLEN + q8, 8), _T)]
                d_row = half * _ROWS_PER_W + it * _FIRE + b
                dst = out_hbm.at[0, head, d_row, :]
                copies.append(pltpu.async_copy(src, dst, sem))
            for c in copies:
                c.wait()
            return carry

        lax.fori_loop(0, _ROWS_PER_W // _FIRE, emit, 0)

    return k(tt_all)


def kernel(T, table):
    j = jnp.arange(_TTL)
    r = jnp.arange(8)
    rows = jnp.clip(j[None, :] + r[:, None] - 24, 0, 2 * _MAXD)
    tt_all = jnp.transpose(table[rows], (0, 2, 1)).reshape(-1)
    return _sc_bias(tt_all)


# speedup vs baseline: 2.4782x; 2.1296x over previous
import functools

import jax
import jax.numpy as jnp
from jax import lax
from jax.experimental import pallas as pl
from jax.experimental.pallas import tpu as pltpu
from jax.experimental.pallas import tpu_sc as plsc

_MAXD = 128
_H = 16
_T = 2048
_WLEN = 4112
_NW = 32
_ROWS_PER_W = (_H * _T) // _NW
_GROUP = 8
_NJT = _T // 128


def _sc_bias(wbuf):
    mesh = plsc.VectorSubcoreMesh(core_axis_name="c", subcore_axis_name="s")

    @functools.partial(
        pl.kernel,
        out_type=jax.ShapeDtypeStruct((_H * (_T // 8) * _NJT, 8, 128), jnp.float32),
        mesh=mesh,
        scratch_types=[
            pltpu.VMEM((_GROUP, _WLEN), jnp.float32),
            pltpu.SemaphoreType.DMA,
        ],
        compiler_params=pltpu.CompilerParams(use_tc_tiling_on_sc=False),
    )
    def k(wbuf_hbm, out_hbm, w_v, sem):
        cid = lax.axis_index("c")
        sid = lax.axis_index("s")
        wid = sid * 2 + cid
        head = wid // 2
        half = wid % 2

        pltpu.sync_copy(wbuf_hbm.at[head], w_v)

        def emit(g, carry):
            q = pl.multiple_of(_T - half * _ROWS_PER_W - _GROUP - g * _GROUP, 8)
            ig = (head * (_T // 8) + half * (_ROWS_PER_W // 8) + g) * _NJT
            copies = []
            for jt in range(_NJT):
                src = w_v.at[:, pl.ds(q + jt * 128, 128)]
                copies.append(pltpu.async_copy(src, out_hbm.at[ig + jt], sem))
            for c in copies:
                c.wait()
            return carry

        lax.fori_loop(0, _ROWS_PER_W // _GROUP, emit, 0)

    return k(wbuf)


def kernel(T, table):
    m = jnp.arange(_WLEN)
    s = jnp.arange(_GROUP)
    p = m[None, :] + (_GROUP - s)[:, None]
    rows = jnp.clip(p - _T, -_MAXD, _MAXD) + _MAXD
    wbuf = jnp.transpose(table[rows], (2, 0, 1))
    tiles = _sc_bias(wbuf)
    out = (
        tiles.reshape(_H, _T // 8, _NJT, 8, 128)
        .transpose(0, 1, 3, 2, 4)
        .reshape(1, _H, _T, _T)
    )
    return out


# speedup vs baseline: 3.7666x; 1.5199x over previous
import functools

import jax
import jax.numpy as jnp
from jax import lax
from jax.experimental import pallas as pl
from jax.experimental.pallas import tpu as pltpu
from jax.experimental.pallas import tpu_sc as plsc

_MAXD = 128
_H = 16
_T = 2048
_WLEN = 4112
_NW = 32
_ROWS_PER_W = (_H * _T) // _NW
_GROUP = 8
_NJT = _T // 128


def _sc_bias(wbuf):
    mesh = plsc.VectorSubcoreMesh(core_axis_name="c", subcore_axis_name="s")

    @functools.partial(
        pl.kernel,
        out_type=jax.ShapeDtypeStruct((_H * (_T // 8) * _NJT, 8, 128), jnp.float32),
        mesh=mesh,
        scratch_types=[
            pltpu.VMEM((_GROUP, _WLEN), jnp.float32),
            pltpu.SemaphoreType.DMA,
        ],
        compiler_params=pltpu.CompilerParams(use_tc_tiling_on_sc=False),
    )
    def k(wbuf_hbm, out_hbm, w_v, sem):
        cid = lax.axis_index("c")
        sid = lax.axis_index("s")
        wid = sid * 2 + cid
        head = wid // 2
        half = wid % 2

        pltpu.sync_copy(wbuf_hbm.at[head], w_v)

        def emit(g, carry):
            q = pl.multiple_of(_T - half * _ROWS_PER_W - _GROUP - g * _GROUP, 8)
            ig = (head * (_T // 8) + half * (_ROWS_PER_W // 8) + g) * _NJT
            copies = []
            for jt in range(_NJT):
                src = w_v.at[:, pl.ds(q + jt * 128, 128)]
                copies.append(pltpu.async_copy(src, out_hbm.at[ig + jt], sem))
            for c in copies:
                c.wait()
            return carry

        lax.fori_loop(0, _ROWS_PER_W // _GROUP, emit, 0)

    return k(wbuf)


def kernel(T, table):
    table_t = table.T
    parts = []
    for s in range(_GROUP):
        c = _GROUP - s
        lo = jnp.broadcast_to(table_t[:, :1], (_H, (_T - _MAXD) - c))
        hi_len = _WLEN - ((_T + _MAXD + 1) - c)
        hi = jnp.broadcast_to(table_t[:, -1:], (_H, hi_len))
        parts.append(jnp.concatenate([lo, table_t, hi], axis=1))
    wbuf = jnp.stack(parts, axis=1)
    tiles = _sc_bias(wbuf)
    out = (
        tiles.reshape(_H, _T // 8, _NJT, 8, 128)
        .transpose(0, 1, 3, 2, 4)
        .reshape(1, _H, _T, _T)
    )
    return out


# speedup vs baseline: 3.9084x; 1.0377x over previous
import functools

import jax
import jax.numpy as jnp
from jax import lax
from jax.experimental import pallas as pl
from jax.experimental.pallas import tpu as pltpu
from jax.experimental.pallas import tpu_sc as plsc

_MAXD = 128
_H = 16
_T = 2048
_WLEN = 4112
_NW = 32
_ROWS_PER_W = (_H * _T) // _NW
_GROUP = 8
_NJT = _T // 128
_TTL = 304
_BAND0 = 1896
_NFILL = (_BAND0 + 8) // 16


def _sc_bias(tt):
    mesh = plsc.VectorSubcoreMesh(core_axis_name="c", subcore_axis_name="s")

    @functools.partial(
        pl.kernel,
        out_type=jax.ShapeDtypeStruct((_H * (_T // 8) * _NJT, 8, 128), jnp.float32),
        mesh=mesh,
        scratch_types=[
            pltpu.VMEM((_GROUP, _WLEN), jnp.float32),
            pltpu.SemaphoreType.DMA,
        ],
        compiler_params=pltpu.CompilerParams(use_tc_tiling_on_sc=False),
    )
    def k(tt_hbm, out_hbm, w_v, sem):
        cid = lax.axis_index("c")
        sid = lax.axis_index("s")
        wid = sid * 2 + cid
        head = wid // 2
        half = wid % 2

        copies = [
            pltpu.async_copy(
                tt_hbm.at[s, head], w_v.at[s, pl.ds(_BAND0, _TTL)], sem
            )
            for s in range(_GROUP)
        ]
        for c in copies:
            c.wait()
        c_lo = [w_v[s, pl.ds(_BAND0, 16)] for s in range(_GROUP)]
        c_hi = [w_v[s, pl.ds(_BAND0 + _TTL - 16, 16)] for s in range(_GROUP)]

        def fill(g, carry):
            off = pl.multiple_of(g * 16, 16)
            for s in range(_GROUP):
                w_v[s, pl.ds(off, 16)] = c_lo[s]
                w_v[s, pl.ds(_BAND0 + 296 + off, 16)] = c_hi[s]
            return carry

        lax.fori_loop(0, _NFILL, fill, 0)

        def emit(g, carry):
            q = pl.multiple_of(_T - half * _ROWS_PER_W - _GROUP - g * _GROUP, 8)
            ig = (head * (_T // 8) + half * (_ROWS_PER_W // 8) + g) * _NJT
            copies = []
            for jt in range(_NJT):
                src = w_v.at[:, pl.ds(q + jt * 128, 128)]
                copies.append(pltpu.async_copy(src, out_hbm.at[ig + jt], sem))
            for c in copies:
                c.wait()
            return carry

        lax.fori_loop(0, _ROWS_PER_W // _GROUP, emit, 0)

    return k(tt)


def kernel(T, table):
    table_t = table.T
    parts = []
    for s in range(_GROUP):
        c = _GROUP - s
        lo = jnp.broadcast_to(table_t[:, :1], (_H, 24 - c))
        hi = jnp.broadcast_to(table_t[:, -1:], (_H, _TTL - 257 - (24 - c)))
        parts.append(jnp.concatenate([lo, table_t, hi], axis=1)[None])
    tt = jnp.concatenate(parts, axis=0)
    tiles = _sc_bias(tt)
    out = (
        tiles.reshape(_H, _T // 8, _NJT, 8, 128)
        .transpose(0, 1, 3, 2, 4)
        .reshape(1, _H, _T, _T)
    )
    return out
